# baseline (speedup 1.0000x reference)
import functools

import jax
import jax.numpy as jnp
from jax import lax
from jax.experimental import pallas as pl
from jax.experimental.pallas import tpu as pltpu
from jax.experimental.pallas import tpu_sc as plsc

N = 4096
E = 65536
HID = 128
K = 10
F = 5
S = 3

NC = 2
NS = 16
NW = NC * NS
RPT = N // NW
HALF = N // NC
C = 512
EN = E + N
EP = EN + C
ACC_ROWS = HALF + 8
L = 16
DW = HID


@functools.cache
def _mesh():
    return plsc.VectorSubcoreMesh(core_axis_name="c", subcore_axis_name="s",
                                  num_cores=NC, num_subcores=NS)


def _sc_deg(dstl, cstarts, zeros_deg, ones_deg):
    def body(dstl_hbm, cst_hbm, zd_hbm, od_hbm, deg_hbm,
             deg_acc, didx, ones_v, cst_v, sem):
        c = lax.axis_index("c")
        s = lax.axis_index("s")
        wid = c * NS + s
        pltpu.sync_copy(cst_hbm.at[wid], cst_v)
        cv = cst_v[...]
        lo = cv[0]
        hi = cv[1]
        pltpu.sync_copy(zd_hbm.at[pl.ds(s * RPT, RPT)],
                        deg_acc.at[pl.ds(s * RPT, RPT)])

        @pl.when(s == 0)
        def _():
            pltpu.sync_copy(zd_hbm.at[pl.ds(HALF, ACC_ROWS - HALF)],
                            deg_acc.at[pl.ds(HALF, ACC_ROWS - HALF)])

        pltpu.sync_copy(od_hbm, ones_v)
        plsc.subcore_barrier()

        def chunk(ci, carry):
            base = ci * C
            pltpu.sync_copy(dstl_hbm.at[pl.ds(base, C)], didx)
            pltpu.sync_copy(ones_v, deg_acc.at[didx], add=True)
            return carry

        lax.fori_loop(lo, hi, chunk, 0)
        plsc.subcore_barrier()

        pltpu.sync_copy(deg_acc.at[pl.ds(s * RPT, RPT)],
                        deg_hbm.at[pl.ds(wid * RPT, RPT)])

    fn = pl.kernel(
        body,
        out_type=jax.ShapeDtypeStruct((N, DW), jnp.float32),
        mesh=_mesh(),
        scratch_types=[
            pltpu.VMEM_SHARED((ACC_ROWS, DW), jnp.float32),
            pltpu.VMEM((C,), jnp.int32),
            pltpu.VMEM((C, DW), jnp.float32),
            pltpu.VMEM((L,), jnp.int32),
            pltpu.SemaphoreType.DMA,
        ],
    )
    return fn(dstl, cstarts, zeros_deg, ones_deg)


def _sc_hop(y, srcs, dstl, cstarts, dinv2, zeros_big):
    def body(y_hbm, srcs_hbm, dstl_hbm, cst_hbm, dv2_hbm, zb_hbm,
             yo_hbm,
             acc, sidx, didx, rows, accv, cst_v, dv2_v, sem):
        c = lax.axis_index("c")
        s = lax.axis_index("s")
        wid = c * NS + s
        pltpu.sync_copy(cst_hbm.at[wid], cst_v)
        cv = cst_v[...]
        lo = cv[0]
        hi = cv[1]
        pltpu.sync_copy(zb_hbm.at[pl.ds(s * RPT, RPT)],
                        acc.at[pl.ds(s * RPT, RPT)])

        @pl.when(s == 0)
        def _():
            pltpu.sync_copy(zb_hbm.at[pl.ds(HALF, ACC_ROWS - HALF)],
                            acc.at[pl.ds(HALF, ACC_ROWS - HALF)])

        pltpu.sync_copy(dv2_hbm.at[pl.ds(wid * RPT, RPT)], dv2_v)
        plsc.subcore_barrier()

        def chunk(ci, carry):
            base = ci * C
            pltpu.sync_copy(srcs_hbm.at[pl.ds(base, C)], sidx)
            pltpu.sync_copy(dstl_hbm.at[pl.ds(base, C)], didx)
            pltpu.async_copy(y_hbm.at[sidx], rows, sem).wait()
            pltpu.sync_copy(rows, acc.at[didx], add=True)
            return carry

        lax.fori_loop(lo, hi, chunk, 0)
        plsc.subcore_barrier()

        pltpu.sync_copy(acc.at[pl.ds(s * RPT, RPT)], accv)

        def rowgrp(rg, carry):
            sf = dv2_v[pl.ds(rg * L, L)]
            for j in range(L):
                sv = lax.broadcast(sf[j], (L,))
                r = rg * L + j
                for cg in range(HID // L):
                    accv[r, pl.ds(cg * L, L)] = accv[r, pl.ds(cg * L, L)] * sv
            return carry

        lax.fori_loop(0, RPT // L, rowgrp, 0)
        pltpu.sync_copy(accv, yo_hbm.at[pl.ds(wid * RPT, RPT)])

    fn = pl.kernel(
        body,
        out_type=jax.ShapeDtypeStruct((N, HID), jnp.float32),
        mesh=_mesh(),
        scratch_types=[
            pltpu.VMEM_SHARED((ACC_ROWS, HID), jnp.float32),
            pltpu.VMEM((C,), jnp.int32),
            pltpu.VMEM((C,), jnp.int32),
            pltpu.VMEM((C, HID), jnp.float32),
            pltpu.VMEM((RPT, HID), jnp.float32),
            pltpu.VMEM((L,), jnp.int32),
            pltpu.VMEM((RPT,), jnp.float32),
            pltpu.SemaphoreType.DMA,
        ],
    )
    return fn(y, srcs, dstl, cstarts, dinv2, zeros_big)


def _prep_edges(edges):
    loops = lax.iota(jnp.int32, N)
    src = jnp.concatenate([edges[:, 0], loops])
    dst = jnp.concatenate([edges[:, 1], loops])
    dst_s, src_s = lax.sort_key_val(dst, src)
    b = jnp.searchsorted(dst_s, jnp.int32(HALF)).astype(jnp.int32)
    shift = (C - b % C) % C
    pos = lax.iota(jnp.int32, EP)
    old = jnp.where(pos < b, pos, pos - shift)
    valid = (pos < b) | ((pos >= b + shift) & (old < EN))
    oldc = jnp.clip(old, 0, EN - 1)
    src_p = jnp.where(valid, src_s[oldc], 0)
    dsel = dst_s[oldc]
    dstl_p = jnp.where(valid, dsel - jnp.where(dsel >= HALF, HALF, 0),
                       jnp.int32(HALF))
    row_bounds = lax.iota(jnp.int32, NW + 1) * RPT
    start = jnp.searchsorted(dst_s, row_bounds).astype(jnp.int32)
    start_p = start + jnp.where(start >= b, shift, 0)
    cs = (start_p + (C - 1)) // C
    cstarts = jnp.zeros((NW, L), jnp.int32)
    cstarts = cstarts.at[:, 0].set(cs[:NW]).at[:, 1].set(cs[1:])
    return src_p, dstl_p, cstarts


def _mlp(feats, degcol, W1, b1, W2, b2):
    BM = 256

    def body(f_ref, d_ref, w1_ref, b1_ref, w2_ref, b2_ref,
             y0_ref, dv2_ref, sqd_ref):
        h = lax.dot_general(f_ref[...], w1_ref[...],
                            (((1,), (0,)), ((), ())),
                            preferred_element_type=jnp.float32)
        h = jnp.maximum(h + b1_ref[...], 0.0)
        o = lax.dot_general(h, w2_ref[...], (((1,), (0,)), ((), ())),
                            preferred_element_type=jnp.float32)
        x = o + b2_ref[...]
        deg = jnp.maximum(d_ref[...], 1.0)
        dinv = lax.rsqrt(deg)
        y0_ref[...] = x * dinv
        dv2_ref[...] = dinv * dinv
        sqd_ref[...] = deg * dinv

    return pl.pallas_call(
        body,
        grid=(N // BM,),
        in_specs=[
            pl.BlockSpec((BM, HID), lambda i: (i, 0)),
            pl.BlockSpec((BM, 1), lambda i: (i, 0)),
            pl.BlockSpec((HID, HID), lambda i: (0, 0)),
            pl.BlockSpec((1, HID), lambda i: (0, 0)),
            pl.BlockSpec((HID, HID), lambda i: (0, 0)),
            pl.BlockSpec((1, HID), lambda i: (0, 0)),
        ],
        out_specs=(
            pl.BlockSpec((BM, HID), lambda i: (i, 0)),
            pl.BlockSpec((BM, 1), lambda i: (i, 0)),
            pl.BlockSpec((BM, 1), lambda i: (i, 0)),
        ),
        out_shape=(
            jax.ShapeDtypeStruct((N, HID), jnp.float32),
            jax.ShapeDtypeStruct((N, 1), jnp.float32),
            jax.ShapeDtypeStruct((N, 1), jnp.float32),
        ),
    )(feats, degcol, W1, b1.reshape(1, HID), W2, b2.reshape(1, HID))


def _filter_attn(ys, sqd, conv_weight, W_filt, W_attn, b_attn):
    BM = 256

    def body(*refs):
        y_refs = refs[: K + 1]
        sd_ref, cw_ref, wf_ref, wa_ref, ba_ref, o_ref = refs[K + 1:]
        sd = sd_ref[...]
        ys_b = [y_refs[k][...] for k in range(K + 1)]
        xb = ys_b[0] * sd
        hs = [[None] * F for _ in range(S)]
        sc = [[None] * F for _ in range(S)]
        for s in range(S):
            wa = wa_ref[s]
            proj = jnp.tanh(
                lax.dot_general(xb, wa, (((1,), (0,)), ((), ())),
                                preferred_element_type=jnp.float32)
                + ba_ref[s][None, :])
            for f in range(F):
                acc = cw_ref[s, f, 0] * ys_b[0]
                for k in range(1, K + 1):
                    acc = acc + cw_ref[s, f, k] * ys_b[k]
                hb = lax.dot_general(acc * sd, wf_ref[f],
                                     (((1,), (0,)), ((), ())),
                                     preferred_element_type=jnp.float32)
                h = jnp.where(hb >= 0.0, hb, 0.01 * hb)
                hs[s][f] = h
                sc[s][f] = jnp.sum(h * proj, axis=1, keepdims=True)
        outs = []
        for s in range(S):
            acc = None
            for f in range(F):
                m = jnp.maximum(jnp.maximum(sc[0][f], sc[1][f]), sc[2][f])
                es = [jnp.exp(sc[t][f] - m) for t in range(S)]
                den = es[0] + es[1] + es[2]
                w_sf = es[s] / den
                t = w_sf * hs[s][f]
                acc = t if acc is None else acc + t
            outs.append(acc)
        o_ref[...] = jnp.stack(outs, axis=0)

    in_specs = [pl.BlockSpec((BM, HID), lambda i: (i, 0))
                for _ in range(K + 1)]
    in_specs += [
        pl.BlockSpec((BM, 1), lambda i: (i, 0)),
        pl.BlockSpec(memory_space=pltpu.MemorySpace.SMEM),
        pl.BlockSpec((F, HID, HID), lambda i: (0, 0, 0)),
        pl.BlockSpec((S, HID, HID), lambda i: (0, 0, 0)),
        pl.BlockSpec((S, HID), lambda i: (0, 0)),
    ]
    return pl.pallas_call(
        body,
        grid=(N // BM,),
        in_specs=in_specs,
        out_specs=pl.BlockSpec((S, BM, HID), lambda i: (0, i, 0)),
        out_shape=jax.ShapeDtypeStruct((S, N, HID), jnp.float32),
    )(*ys, sqd, conv_weight, W_filt, W_attn, b_attn)


def _recons(h_):
    BM = 512

    def body(a_ref, b_ref, o_ref):
        r = lax.dot_general(a_ref[0], b_ref[0], (((1,), (1,)), ((), ())),
                            preferred_element_type=jnp.float32)
        o_ref[0] = 1.0 / (1.0 + jnp.exp(-r))

    return pl.pallas_call(
        body,
        grid=(S, N // BM, N // BM),
        in_specs=[
            pl.BlockSpec((1, BM, HID), lambda s, i, j: (s, i, 0)),
            pl.BlockSpec((1, BM, HID), lambda s, i, j: (s, j, 0)),
        ],
        out_specs=pl.BlockSpec((1, BM, BM), lambda s, i, j: (s, i, j)),
        out_shape=jax.ShapeDtypeStruct((S, N, N), jnp.float32),
    )(h_, h_)


def kernel(edges, feats, W1, b1, W2, b2, conv_weight, W_filt, W_attn, b_attn):
    srcs, dstl, cstarts = _prep_edges(edges)
    zeros_big = jnp.zeros((ACC_ROWS, HID), jnp.float32)
    ones_deg = jnp.ones((C, DW), jnp.float32)
    deg = _sc_deg(dstl, cstarts, zeros_big, ones_deg)
    y, dv2, sqd = _mlp(feats, deg[:, :1], W1, b1, W2, b2)
    dinv2 = dv2.reshape(N)

    def hop_step(cur, _):
        nxt = _sc_hop(cur, srcs, dstl, cstarts, dinv2, zeros_big)
        return nxt, nxt

    _, ys_stack = lax.scan(hop_step, y, None, length=K)
    ys = [y] + [ys_stack[k] for k in range(K)]
    h_ = _filter_attn(ys, sqd, conv_weight, W_filt, W_attn, b_attn)
    r = _recons(h_)
    return r, r, h_


# speedup vs baseline: 1.0405x; 1.0405x over previous
import functools

import jax
import jax.numpy as jnp
from jax import lax
from jax.experimental import pallas as pl
from jax.experimental.pallas import tpu as pltpu
from jax.experimental.pallas import tpu_sc as plsc

N = 4096
E = 65536
HID = 128
K = 10
F = 5
S = 3

NC = 2
NS = 16
NW = NC * NS
RPT = N // NW
HALF = N // NC
C = 128
EN = E + N
EP = EN + C
ACC_ROWS = HALF + 8
L = 16
DW = HID


@functools.cache
def _mesh():
    return plsc.VectorSubcoreMesh(core_axis_name="c", subcore_axis_name="s",
                                  num_cores=NC, num_subcores=NS)


def _sc_deg(dstl, cstarts, zeros_deg, ones_deg):
    def body(dstl_hbm, cst_hbm, zd_hbm, od_hbm, deg_hbm,
             deg_acc, didx, ones_v, cst_v, sem):
        c = lax.axis_index("c")
        s = lax.axis_index("s")
        wid = c * NS + s
        pltpu.sync_copy(cst_hbm.at[wid], cst_v)
        cv = cst_v[...]
        lo = cv[0]
        hi = cv[1]
        pltpu.sync_copy(zd_hbm.at[pl.ds(s * RPT, RPT)],
                        deg_acc.at[pl.ds(s * RPT, RPT)])

        @pl.when(s == 0)
        def _():
            pltpu.sync_copy(zd_hbm.at[pl.ds(HALF, ACC_ROWS - HALF)],
                            deg_acc.at[pl.ds(HALF, ACC_ROWS - HALF)])

        pltpu.sync_copy(od_hbm, ones_v)
        plsc.subcore_barrier()

        def chunk(ci, carry):
            base = ci * C
            pltpu.sync_copy(dstl_hbm.at[pl.ds(base, C)], didx)
            pltpu.sync_copy(ones_v, deg_acc.at[didx], add=True)
            return carry

        lax.fori_loop(lo, hi, chunk, 0)
        plsc.subcore_barrier()

        pltpu.sync_copy(deg_acc.at[pl.ds(s * RPT, RPT)],
                        deg_hbm.at[pl.ds(wid * RPT, RPT)])

    fn = pl.kernel(
        body,
        out_type=jax.ShapeDtypeStruct((N, DW), jnp.float32),
        mesh=_mesh(),
        scratch_types=[
            pltpu.VMEM_SHARED((ACC_ROWS, DW), jnp.float32),
            pltpu.VMEM((C,), jnp.int32),
            pltpu.VMEM((C, DW), jnp.float32),
            pltpu.VMEM((L,), jnp.int32),
            pltpu.SemaphoreType.DMA,
        ],
    )
    return fn(dstl, cstarts, zeros_deg, ones_deg)


def _sc_hop(y, srcs, dstl, cstarts, dinv2, zeros_big):
    def body(y_hbm, srcs_hbm, dstl_hbm, cst_hbm, dv2_hbm, zb_hbm,
             yo_hbm,
             acc, sidx, didx, rows, accv, cst_v, dv2_v, sem):
        c = lax.axis_index("c")
        s = lax.axis_index("s")
        wid = c * NS + s
        pltpu.sync_copy(cst_hbm.at[wid], cst_v)
        cv = cst_v[...]
        lo = cv[0]
        hi = cv[1]
        pltpu.sync_copy(zb_hbm.at[pl.ds(s * RPT, RPT)],
                        acc.at[pl.ds(s * RPT, RPT)])

        @pl.when(s == 0)
        def _():
            pltpu.sync_copy(zb_hbm.at[pl.ds(HALF, ACC_ROWS - HALF)],
                            acc.at[pl.ds(HALF, ACC_ROWS - HALF)])

        pltpu.sync_copy(dv2_hbm.at[pl.ds(wid * RPT, RPT)], dv2_v)
        plsc.subcore_barrier()

        def chunk(ci, carry):
            base = ci * C
            pltpu.sync_copy(srcs_hbm.at[pl.ds(base, C)], sidx)
            pltpu.sync_copy(dstl_hbm.at[pl.ds(base, C)], didx)
            pltpu.async_copy(y_hbm.at[sidx], rows, sem).wait()
            pltpu.sync_copy(rows, acc.at[didx], add=True)
            return carry

        lax.fori_loop(lo, hi, chunk, 0)
        plsc.subcore_barrier()

        pltpu.sync_copy(acc.at[pl.ds(s * RPT, RPT)], accv)

        def rowgrp(rg, carry):
            sf = dv2_v[pl.ds(rg * L, L)]
            for j in range(L):
                sv = lax.broadcast(sf[j], (L,))
                r = rg * L + j
                for cg in range(HID // L):
                    accv[r, pl.ds(cg * L, L)] = accv[r, pl.ds(cg * L, L)] * sv
            return carry

        lax.fori_loop(0, RPT // L, rowgrp, 0)
        pltpu.sync_copy(accv, yo_hbm.at[pl.ds(wid * RPT, RPT)])

    fn = pl.kernel(
        body,
        out_type=jax.ShapeDtypeStruct((N, HID), jnp.float32),
        mesh=_mesh(),
        scratch_types=[
            pltpu.VMEM_SHARED((ACC_ROWS, HID), jnp.float32),
            pltpu.VMEM((C,), jnp.int32),
            pltpu.VMEM((C,), jnp.int32),
            pltpu.VMEM((C, HID), jnp.float32),
            pltpu.VMEM((RPT, HID), jnp.float32),
            pltpu.VMEM((L,), jnp.int32),
            pltpu.VMEM((RPT,), jnp.float32),
            pltpu.SemaphoreType.DMA,
        ],
    )
    return fn(y, srcs, dstl, cstarts, dinv2, zeros_big)


def _prep_edges(edges):
    loops = lax.iota(jnp.int32, N)
    src = jnp.concatenate([edges[:, 0], loops])
    dst = jnp.concatenate([edges[:, 1], loops])
    dst_s, src_s = lax.sort_key_val(dst, src)
    b = jnp.searchsorted(dst_s, jnp.int32(HALF)).astype(jnp.int32)
    shift = (C - b % C) % C
    pos = lax.iota(jnp.int32, EP)
    old = jnp.where(pos < b, pos, pos - shift)
    valid = (pos < b) | ((pos >= b + shift) & (old < EN))
    oldc = jnp.clip(old, 0, EN - 1)
    src_p = jnp.where(valid, src_s[oldc], 0)
    dsel = dst_s[oldc]
    dstl_p = jnp.where(valid, dsel - jnp.where(dsel >= HALF, HALF, 0),
                       jnp.int32(HALF))
    row_bounds = lax.iota(jnp.int32, NW + 1) * RPT
    start = jnp.searchsorted(dst_s, row_bounds).astype(jnp.int32)
    start_p = start + jnp.where(start >= b, shift, 0)
    cs = (start_p + (C - 1)) // C
    cstarts = jnp.zeros((NW, L), jnp.int32)
    cstarts = cstarts.at[:, 0].set(cs[:NW]).at[:, 1].set(cs[1:])
    return src_p, dstl_p, cstarts


def _mlp(feats, degcol, W1, b1, W2, b2):
    BM = 256

    def body(f_ref, d_ref, w1_ref, b1_ref, w2_ref, b2_ref,
             y0_ref, dv2_ref, sqd_ref):
        h = lax.dot_general(f_ref[...], w1_ref[...],
                            (((1,), (0,)), ((), ())),
                            preferred_element_type=jnp.float32)
        h = jnp.maximum(h + b1_ref[...], 0.0)
        o = lax.dot_general(h, w2_ref[...], (((1,), (0,)), ((), ())),
                            preferred_element_type=jnp.float32)
        x = o + b2_ref[...]
        deg = jnp.maximum(d_ref[...], 1.0)
        dinv = lax.rsqrt(deg)
        y0_ref[...] = x * dinv
        dv2_ref[...] = dinv * dinv
        sqd_ref[...] = deg * dinv

    return pl.pallas_call(
        body,
        grid=(N // BM,),
        in_specs=[
            pl.BlockSpec((BM, HID), lambda i: (i, 0)),
            pl.BlockSpec((BM, 1), lambda i: (i, 0)),
            pl.BlockSpec((HID, HID), lambda i: (0, 0)),
            pl.BlockSpec((1, HID), lambda i: (0, 0)),
            pl.BlockSpec((HID, HID), lambda i: (0, 0)),
            pl.BlockSpec((1, HID), lambda i: (0, 0)),
        ],
        out_specs=(
            pl.BlockSpec((BM, HID), lambda i: (i, 0)),
            pl.BlockSpec((BM, 1), lambda i: (i, 0)),
            pl.BlockSpec((BM, 1), lambda i: (i, 0)),
        ),
        out_shape=(
            jax.ShapeDtypeStruct((N, HID), jnp.float32),
            jax.ShapeDtypeStruct((N, 1), jnp.float32),
            jax.ShapeDtypeStruct((N, 1), jnp.float32),
        ),
    )(feats, degcol, W1, b1.reshape(1, HID), W2, b2.reshape(1, HID))


def _filter_attn(ys, sqd, conv_weight, W_filt, W_attn, b_attn):
    BM = 256

    def body(*refs):
        y_refs = refs[: K + 1]
        sd_ref, cw_ref, wf_ref, wa_ref, ba_ref, o_ref = refs[K + 1:]
        sd = sd_ref[...]
        ys_b = [y_refs[k][...] for k in range(K + 1)]
        xb = ys_b[0] * sd
        hs = [[None] * F for _ in range(S)]
        sc = [[None] * F for _ in range(S)]
        for s in range(S):
            wa = wa_ref[s]
            proj = jnp.tanh(
                lax.dot_general(xb, wa, (((1,), (0,)), ((), ())),
                                preferred_element_type=jnp.float32)
                + ba_ref[s][None, :])
            for f in range(F):
                acc = cw_ref[s, f, 0] * ys_b[0]
                for k in range(1, K + 1):
                    acc = acc + cw_ref[s, f, k] * ys_b[k]
                hb = lax.dot_general(acc * sd, wf_ref[f],
                                     (((1,), (0,)), ((), ())),
                                     preferred_element_type=jnp.float32)
                h = jnp.where(hb >= 0.0, hb, 0.01 * hb)
                hs[s][f] = h
                sc[s][f] = jnp.sum(h * proj, axis=1, keepdims=True)
        outs = []
        for s in range(S):
            acc = None
            for f in range(F):
                m = jnp.maximum(jnp.maximum(sc[0][f], sc[1][f]), sc[2][f])
                es = [jnp.exp(sc[t][f] - m) for t in range(S)]
                den = es[0] + es[1] + es[2]
                w_sf = es[s] / den
                t = w_sf * hs[s][f]
                acc = t if acc is None else acc + t
            outs.append(acc)
        o_ref[...] = jnp.stack(outs, axis=0)

    in_specs = [pl.BlockSpec((BM, HID), lambda i: (i, 0))
                for _ in range(K + 1)]
    in_specs += [
        pl.BlockSpec((BM, 1), lambda i: (i, 0)),
        pl.BlockSpec(memory_space=pltpu.MemorySpace.SMEM),
        pl.BlockSpec((F, HID, HID), lambda i: (0, 0, 0)),
        pl.BlockSpec((S, HID, HID), lambda i: (0, 0, 0)),
        pl.BlockSpec((S, HID), lambda i: (0, 0)),
    ]
    return pl.pallas_call(
        body,
        grid=(N // BM,),
        in_specs=in_specs,
        out_specs=pl.BlockSpec((S, BM, HID), lambda i: (0, i, 0)),
        out_shape=jax.ShapeDtypeStruct((S, N, HID), jnp.float32),
    )(*ys, sqd, conv_weight, W_filt, W_attn, b_attn)


def _recons(h_):
    BM = 512

    def body(a_ref, b_ref, o_ref):
        a = a_ref[0].astype(jnp.bfloat16)
        b = b_ref[0].astype(jnp.bfloat16)
        r = lax.dot_general(a, b, (((1,), (1,)), ((), ())),
                            preferred_element_type=jnp.float32)
        o_ref[0] = 1.0 / (1.0 + jnp.exp(-r))

    return pl.pallas_call(
        body,
        grid=(S, N // BM, N // BM),
        in_specs=[
            pl.BlockSpec((1, BM, HID), lambda s, i, j: (s, i, 0)),
            pl.BlockSpec((1, BM, HID), lambda s, i, j: (s, j, 0)),
        ],
        out_specs=pl.BlockSpec((1, BM, BM), lambda s, i, j: (s, i, j)),
        out_shape=jax.ShapeDtypeStruct((S, N, N), jnp.float32),
    )(h_, h_)


def kernel(edges, feats, W1, b1, W2, b2, conv_weight, W_filt, W_attn, b_attn):
    srcs, dstl, cstarts = _prep_edges(edges)
    zeros_big = jnp.zeros((ACC_ROWS, HID), jnp.float32)
    ones_deg = jnp.ones((C, DW), jnp.float32)
    deg = _sc_deg(dstl, cstarts, zeros_big, ones_deg)
    y, dv2, sqd = _mlp(feats, deg[:, :1], W1, b1, W2, b2)
    dinv2 = dv2.reshape(N)

    def hop_step(cur, _):
        nxt = _sc_hop(cur, srcs, dstl, cstarts, dinv2, zeros_big)
        return nxt, nxt

    _, ys_stack = lax.scan(hop_step, y, None, length=K)
    ys = [y] + [ys_stack[k] for k in range(K)]
    h_ = _filter_attn(ys, sqd, conv_weight, W_filt, W_attn, b_attn)
    r = _recons(h_)
    return r, r, h_


# speedup vs baseline: 1.1302x; 1.0862x over previous
import functools

import jax
import jax.numpy as jnp
from jax import lax
from jax.experimental import pallas as pl
from jax.experimental.pallas import tpu as pltpu
from jax.experimental.pallas import tpu_sc as plsc

N = 4096
E = 65536
HID = 128
K = 10
F = 5
S = 3

NC = 2
NS = 16
NW = NC * NS
RPT = N // NW
HALF = N // NC
C = 128
EN = E + N
EP = EN + C
ACC_ROWS = HALF + 8
L = 16
DW = HID


@functools.cache
def _mesh():
    return plsc.VectorSubcoreMesh(core_axis_name="c", subcore_axis_name="s",
                                  num_cores=NC, num_subcores=NS)


def _sc_deg(dstl, cstarts, zeros_deg, ones_deg):
    def body(dstl_hbm, cst_hbm, zd_hbm, od_hbm, deg_hbm,
             deg_acc, didx, ones_v, cst_v, sem):
        c = lax.axis_index("c")
        s = lax.axis_index("s")
        wid = c * NS + s
        pltpu.sync_copy(cst_hbm.at[wid], cst_v)
        cv = cst_v[...]
        lo = cv[0]
        hi = cv[1]
        pltpu.sync_copy(zd_hbm.at[pl.ds(s * RPT, RPT)],
                        deg_acc.at[pl.ds(s * RPT, RPT)])

        @pl.when(s == 0)
        def _():
            pltpu.sync_copy(zd_hbm.at[pl.ds(HALF, ACC_ROWS - HALF)],
                            deg_acc.at[pl.ds(HALF, ACC_ROWS - HALF)])

        pltpu.sync_copy(od_hbm, ones_v)
        plsc.subcore_barrier()

        def chunk(ci, carry):
            base = ci * C
            pltpu.sync_copy(dstl_hbm.at[pl.ds(base, C)], didx)
            pltpu.sync_copy(ones_v, deg_acc.at[didx], add=True)
            return carry

        lax.fori_loop(lo, hi, chunk, 0)
        plsc.subcore_barrier()

        pltpu.sync_copy(deg_acc.at[pl.ds(s * RPT, RPT)],
                        deg_hbm.at[pl.ds(wid * RPT, RPT)])

    fn = pl.kernel(
        body,
        out_type=jax.ShapeDtypeStruct((N, DW), jnp.float32),
        mesh=_mesh(),
        scratch_types=[
            pltpu.VMEM_SHARED((ACC_ROWS, DW), jnp.float32),
            pltpu.VMEM((C,), jnp.int32),
            pltpu.VMEM((C, DW), jnp.float32),
            pltpu.VMEM((L,), jnp.int32),
            pltpu.SemaphoreType.DMA,
        ],
    )
    return fn(dstl, cstarts, zeros_deg, ones_deg)


def _sc_hop(y, srcs, dstl, cstarts, dinv2, zeros_big):
    def body(y_hbm, srcs_hbm, dstl_hbm, cst_hbm, dv2_hbm, zb_hbm,
             yo_hbm,
             acc, sidx, didx, rows, accv, cst_v, dv2_v, sem):
        c = lax.axis_index("c")
        s = lax.axis_index("s")
        wid = c * NS + s
        pltpu.sync_copy(cst_hbm.at[wid], cst_v)
        cv = cst_v[...]
        lo = cv[0]
        hi = cv[1]
        pltpu.sync_copy(zb_hbm.at[pl.ds(s * RPT, RPT)],
                        acc.at[pl.ds(s * RPT, RPT)])

        @pl.when(s == 0)
        def _():
            pltpu.sync_copy(zb_hbm.at[pl.ds(HALF, ACC_ROWS - HALF)],
                            acc.at[pl.ds(HALF, ACC_ROWS - HALF)])

        pltpu.sync_copy(dv2_hbm.at[pl.ds(wid * RPT, RPT)], dv2_v)
        plsc.subcore_barrier()

        def chunk(ci, carry):
            base = ci * C
            pltpu.sync_copy(srcs_hbm.at[pl.ds(base, C)], sidx)
            pltpu.sync_copy(dstl_hbm.at[pl.ds(base, C)], didx)
            pltpu.async_copy(y_hbm.at[sidx], rows, sem).wait()
            pltpu.sync_copy(rows, acc.at[didx], add=True)
            return carry

        lax.fori_loop(lo, hi, chunk, 0)
        plsc.subcore_barrier()

        pltpu.sync_copy(acc.at[pl.ds(s * RPT, RPT)], accv)

        def rowgrp(rg, carry):
            sf = dv2_v[pl.ds(rg * L, L)]
            for j in range(L):
                sv = lax.broadcast(sf[j], (L,))
                r = rg * L + j
                for cg in range(HID // L):
                    accv[r, pl.ds(cg * L, L)] = accv[r, pl.ds(cg * L, L)] * sv
            return carry

        lax.fori_loop(0, RPT // L, rowgrp, 0)
        pltpu.sync_copy(accv, yo_hbm.at[pl.ds(wid * RPT, RPT)])

    fn = pl.kernel(
        body,
        out_type=jax.ShapeDtypeStruct((N, HID), jnp.float32),
        mesh=_mesh(),
        scratch_types=[
            pltpu.VMEM_SHARED((ACC_ROWS, HID), jnp.float32),
            pltpu.VMEM((C,), jnp.int32),
            pltpu.VMEM((C,), jnp.int32),
            pltpu.VMEM((C, HID), jnp.float32),
            pltpu.VMEM((RPT, HID), jnp.float32),
            pltpu.VMEM((L,), jnp.int32),
            pltpu.VMEM((RPT,), jnp.float32),
            pltpu.SemaphoreType.DMA,
        ],
    )
    return fn(y, srcs, dstl, cstarts, dinv2, zeros_big)


def _prep_edges(edges):
    loops = lax.iota(jnp.int32, N)
    src = jnp.concatenate([edges[:, 0], loops])
    dst = jnp.concatenate([edges[:, 1], loops])
    dst_s, src_s = lax.sort_key_val(dst, src)
    b = jnp.searchsorted(dst_s, jnp.int32(HALF)).astype(jnp.int32)
    shift = (C - b % C) % C
    pos = lax.iota(jnp.int32, EP)
    old = jnp.where(pos < b, pos, pos - shift)
    valid = (pos < b) | ((pos >= b + shift) & (old < EN))
    oldc = jnp.clip(old, 0, EN - 1)
    src_p = jnp.where(valid, src_s[oldc], 0)
    dsel = dst_s[oldc]
    dstl_p = jnp.where(valid, dsel - jnp.where(dsel >= HALF, HALF, 0),
                       jnp.int32(HALF))
    row_bounds = lax.iota(jnp.int32, NW + 1) * RPT
    start = jnp.searchsorted(dst_s, row_bounds).astype(jnp.int32)
    start_p = start + jnp.where(start >= b, shift, 0)
    cs = (start_p + (C - 1)) // C
    cstarts = jnp.zeros((NW, L), jnp.int32)
    cstarts = cstarts.at[:, 0].set(cs[:NW]).at[:, 1].set(cs[1:])
    return src_p, dstl_p, cstarts


def _mlp(feats, degcol, W1, b1, W2, b2):
    BM = 256

    def body(f_ref, d_ref, w1_ref, b1_ref, w2_ref, b2_ref,
             y0_ref, dv2_ref, sqd_ref):
        h = lax.dot_general(f_ref[...], w1_ref[...],
                            (((1,), (0,)), ((), ())),
                            preferred_element_type=jnp.float32)
        h = jnp.maximum(h + b1_ref[...], 0.0)
        o = lax.dot_general(h, w2_ref[...], (((1,), (0,)), ((), ())),
                            preferred_element_type=jnp.float32)
        x = o + b2_ref[...]
        deg = jnp.maximum(d_ref[...], 1.0)
        dinv = lax.rsqrt(deg)
        y0_ref[...] = x * dinv
        dv2_ref[...] = dinv * dinv
        sqd_ref[...] = deg * dinv

    return pl.pallas_call(
        body,
        grid=(N // BM,),
        in_specs=[
            pl.BlockSpec((BM, HID), lambda i: (i, 0)),
            pl.BlockSpec((BM, 1), lambda i: (i, 0)),
            pl.BlockSpec((HID, HID), lambda i: (0, 0)),
            pl.BlockSpec((1, HID), lambda i: (0, 0)),
            pl.BlockSpec((HID, HID), lambda i: (0, 0)),
            pl.BlockSpec((1, HID), lambda i: (0, 0)),
        ],
        out_specs=(
            pl.BlockSpec((BM, HID), lambda i: (i, 0)),
            pl.BlockSpec((BM, 1), lambda i: (i, 0)),
            pl.BlockSpec((BM, 1), lambda i: (i, 0)),
        ),
        out_shape=(
            jax.ShapeDtypeStruct((N, HID), jnp.float32),
            jax.ShapeDtypeStruct((N, 1), jnp.float32),
            jax.ShapeDtypeStruct((N, 1), jnp.float32),
        ),
    )(feats, degcol, W1, b1.reshape(1, HID), W2, b2.reshape(1, HID))


def _filter_attn(ys, sqd, conv_weight, W_filt, W_attn, b_attn):
    BM = 256

    def body(*refs):
        y_refs = refs[: K + 1]
        sd_ref, cw_ref, wf_ref, wa_ref, ba_ref, o_ref = refs[K + 1:]
        sd = sd_ref[...]
        ys_b = [y_refs[k][...] for k in range(K + 1)]
        xb = ys_b[0] * sd
        hs = [[None] * F for _ in range(S)]
        sc = [[None] * F for _ in range(S)]
        for s in range(S):
            wa = wa_ref[s]
            proj = jnp.tanh(
                lax.dot_general(xb, wa, (((1,), (0,)), ((), ())),
                                preferred_element_type=jnp.float32)
                + ba_ref[s][None, :])
            for f in range(F):
                acc = cw_ref[s, f, 0] * ys_b[0]
                for k in range(1, K + 1):
                    acc = acc + cw_ref[s, f, k] * ys_b[k]
                hb = lax.dot_general(acc * sd, wf_ref[f],
                                     (((1,), (0,)), ((), ())),
                                     preferred_element_type=jnp.float32)
                h = jnp.where(hb >= 0.0, hb, 0.01 * hb)
                hs[s][f] = h
                sc[s][f] = jnp.sum(h * proj, axis=1, keepdims=True)
        outs = []
        for s in range(S):
            acc = None
            for f in range(F):
                m = jnp.maximum(jnp.maximum(sc[0][f], sc[1][f]), sc[2][f])
                es = [jnp.exp(sc[t][f] - m) for t in range(S)]
                den = es[0] + es[1] + es[2]
                w_sf = es[s] / den
                t = w_sf * hs[s][f]
                acc = t if acc is None else acc + t
            outs.append(acc)
        o_ref[...] = jnp.stack(outs, axis=0)

    in_specs = [pl.BlockSpec((BM, HID), lambda i: (i, 0))
                for _ in range(K + 1)]
    in_specs += [
        pl.BlockSpec((BM, 1), lambda i: (i, 0)),
        pl.BlockSpec(memory_space=pltpu.MemorySpace.SMEM),
        pl.BlockSpec((F, HID, HID), lambda i: (0, 0, 0)),
        pl.BlockSpec((S, HID, HID), lambda i: (0, 0, 0)),
        pl.BlockSpec((S, HID), lambda i: (0, 0)),
    ]
    return pl.pallas_call(
        body,
        grid=(N // BM,),
        in_specs=in_specs,
        out_specs=pl.BlockSpec((S, BM, HID), lambda i: (0, i, 0)),
        out_shape=jax.ShapeDtypeStruct((S, N, HID), jnp.float32),
    )(*ys, sqd, conv_weight, W_filt, W_attn, b_attn)


def _recons(h_):
    BM = 1024

    def body(a_ref, b_ref, o_ref):
        a = a_ref[0].astype(jnp.bfloat16)
        b = b_ref[0].astype(jnp.bfloat16)
        r = lax.dot_general(a, b, (((1,), (1,)), ((), ())),
                            preferred_element_type=jnp.float32)
        o_ref[0] = 1.0 / (1.0 + jnp.exp(-r))

    return pl.pallas_call(
        body,
        grid=(S, N // BM, N // BM),
        in_specs=[
            pl.BlockSpec((1, BM, HID), lambda s, i, j: (s, i, 0)),
            pl.BlockSpec((1, BM, HID), lambda s, i, j: (s, j, 0)),
        ],
        out_specs=pl.BlockSpec((1, BM, BM), lambda s, i, j: (s, i, j)),
        out_shape=jax.ShapeDtypeStruct((S, N, N), jnp.float32),
    )(h_, h_)


def kernel(edges, feats, W1, b1, W2, b2, conv_weight, W_filt, W_attn, b_attn):
    srcs, dstl, cstarts = _prep_edges(edges)
    zeros_big = jnp.zeros((ACC_ROWS, HID), jnp.float32)
    ones_deg = jnp.ones((C, DW), jnp.float32)
    deg = _sc_deg(dstl, cstarts, zeros_big, ones_deg)
    y, dv2, sqd = _mlp(feats, deg[:, :1], W1, b1, W2, b2)
    dinv2 = dv2.reshape(N)

    def hop_step(cur, _):
        nxt = _sc_hop(cur, srcs, dstl, cstarts, dinv2, zeros_big)
        return nxt, nxt

    _, ys_stack = lax.scan(hop_step, y, None, length=K)
    ys = [y] + [ys_stack[k] for k in range(K)]
    h_ = _filter_attn(ys, sqd, conv_weight, W_filt, W_attn, b_attn)
    r = _recons(h_)
    return r, r, h_
